# baseline (speedup 1.0000x reference)
import jax
import jax.numpy as jnp
from jax import lax
from jax.experimental import pallas as pl
from jax.experimental.pallas import tpu as pltpu
from jax.experimental.pallas import tpu_sc as plsc

B = 4096
HIST = 200
D = 256
NCLASS = 1000

NC = 2
NS = 16
NW = NC * NS
ROWS_PER_W = B // NW
HIST_PAD = 208
CHUNK = HIST_PAD // 2
NCHUNK = 2 * ROWS_PER_W
DC = D // 16
NGB = 4
NIB = 8


def _pool_body(x_hbm, table_hbm, h0_hbm, bufs, idxs, stage, gsems, isems):
    wid = lax.axis_index("s") * NC + lax.axis_index("c")

    def fire_idx(c, i):
        pltpu.async_copy(x_hbm.at[wid, c], idxs[i], isems[i])

    def wait_idx(i):
        pltpu.make_async_copy(x_hbm.at[0, 0], idxs[i], isems[i]).wait()

    def fire_gather(i, j):
        pltpu.async_copy(table_hbm.at[idxs[i]], bufs[j], gsems[j])

    def wait_gather(j):
        pltpu.make_async_copy(table_hbm.at[pl.ds(0, CHUNK)], bufs[j], gsems[j]).wait()

    def reduce_chunk(buf, carry):
        def body(l, carry):
            sums, cnts = carry
            new_s, new_c = [], []
            for dc in range(DC):
                v = buf[l, pl.ds(dc * 16, 16)]
                bits = lax.bitcast_convert_type(v, jnp.uint32)
                nz = jnp.minimum(bits + bits, jnp.uint32(1))
                new_s.append(sums[dc] + v)
                new_c.append(cnts[dc] + nz)
            return tuple(new_s), tuple(new_c)
        return lax.fori_loop(0, CHUNK, body, carry)

    zf = tuple(jnp.zeros((16,), jnp.float32) for _ in range(DC))
    zu = tuple(jnp.zeros((16,), jnp.uint32) for _ in range(DC))

    for c in range(NIB):
        fire_idx(c, c)
    for c in range(NGB):
        wait_idx(c)
        fire_gather(c, c)

    def iter_body(t, _):
        c0 = 8 * t
        carry = (zf, zu)
        for q in range(8):
            j = q % 4
            wait_gather(j)
            if q % 2 == 0:
                carry = (zf, zu)
            carry = reduce_chunk(bufs[j], carry)

            @pl.when(c0 + q + NGB < NCHUNK)
            def _(q=q, j=j):
                wait_idx((q + NGB) % NIB)
                fire_gather((q + NGB) % NIB, j)

            @pl.when(c0 + q + NIB < NCHUNK)
            def _(q=q):
                fire_idx(c0 + q + NIB, q)

            if q % 2 == 1:
                sums, cnts = carry
                row4 = q // 2 + 4 * lax.rem(t, 2)
                for dc in range(DC):
                    denom = jnp.maximum(cnts[dc].astype(jnp.float32), 1.0)
                    stage[row4, pl.ds(dc * 16, 16)] = sums[dc] / denom

        @pl.when(lax.rem(t, 2) == 1)
        def _():
            base = pl.multiple_of(wid * ROWS_PER_W + 4 * t - 4, 8)
            pltpu.sync_copy(stage, h0_hbm.at[pl.ds(base, 8)])
        return 0

    lax.fori_loop(0, NCHUNK // 8, iter_body, 0)


def _pool(x_r, table):
    return pl.kernel(
        _pool_body,
        mesh=plsc.VectorSubcoreMesh(core_axis_name="c", subcore_axis_name="s"),
        compiler_params=pltpu.CompilerParams(use_tc_tiling_on_sc=False),
        out_type=jax.ShapeDtypeStruct((B, D), jnp.float32),
        scratch_types=[
            [pltpu.VMEM((CHUNK, D), jnp.float32) for _ in range(NGB)],
            [pltpu.VMEM((CHUNK,), jnp.int32) for _ in range(NIB)],
            pltpu.VMEM((8, D), jnp.float32),
            [pltpu.SemaphoreType.DMA for _ in range(NGB)],
            [pltpu.SemaphoreType.DMA for _ in range(NIB)],
        ],
    )(x_r, table)


def _mlp_body(h0_ref, W1_ref, b1_ref, W2_ref, b2_ref, out_ref):
    h1 = jnp.dot(h0_ref[...], W1_ref[...], preferred_element_type=jnp.float32)
    h1 = jnp.maximum(h1 + b1_ref[...], 0.0)
    o = jnp.dot(h1, W2_ref[...], preferred_element_type=jnp.float32)
    out_ref[...] = jnp.maximum(o + b2_ref[...], 0.0)


def _mlp(h0, W1, b1, W2, b2):
    bm = 512
    return pl.pallas_call(
        _mlp_body,
        grid=(B // bm,),
        in_specs=[
            pl.BlockSpec((bm, D), lambda i: (i, 0)),
            pl.BlockSpec((D, 128), lambda i: (0, 0)),
            pl.BlockSpec((1, 128), lambda i: (0, 0)),
            pl.BlockSpec((128, NCLASS), lambda i: (0, 0)),
            pl.BlockSpec((1, NCLASS), lambda i: (0, 0)),
        ],
        out_specs=pl.BlockSpec((bm, NCLASS), lambda i: (i, 0)),
        out_shape=jax.ShapeDtypeStruct((B, NCLASS), jnp.float32),
    )(h0, W1, b1.reshape(1, -1), W2, b2.reshape(1, -1))


def kernel(x, table, W1, b1, W2, b2):
    x_pad = jnp.pad(x, ((0, 0), (0, HIST_PAD - HIST)))
    x_r = x_pad.reshape(NW, NCHUNK, CHUNK)
    h0 = _pool(x_r, table)
    return _mlp(h0, W1, b1, W2, b2)


# speedup vs baseline: 6.3851x; 6.3851x over previous
import jax
import jax.numpy as jnp
from jax import lax
from jax.experimental import pallas as pl
from jax.experimental.pallas import tpu as pltpu
from jax.experimental.pallas import tpu_sc as plsc

B = 4096
HIST = 200
D = 256
NCLASS = 1000

NC = 2
NS = 16
NW = NC * NS
ROWS_PER_W = B // NW
CHA = 104
CHB = HIST - CHA
DC = D // 16
NIB = 4


def _pool_body(xa_hbm, xb_hbm, table_hbm, h0_hbm,
               bufa, bufb, idxa, idxb, stage, gsa, gsb, isa, isb):
    wid = lax.axis_index("s") * NC + lax.axis_index("c")

    def wait_idx(x_hbm, k, ring, sems):
        pltpu.make_async_copy(x_hbm.at[0, 0], ring[k], sems[k]).wait()

    def fire_gather(k, buf, sem, ring):
        pltpu.async_copy(table_hbm.at[ring[k]], buf, sem)

    def wait_gather(n, buf, sem):
        pltpu.make_async_copy(table_hbm.at[pl.ds(0, n)], buf, sem).wait()

    def reduce_chunk(buf, n, carry):
        def body(l, carry):
            sums, cnts = carry
            new_s, new_c = [], []
            for dc in range(DC):
                v = buf[l, pl.ds(dc * 16, 16)]
                bits = lax.bitcast_convert_type(v, jnp.uint32)
                nz = jnp.minimum(bits + bits, jnp.uint32(1))
                new_s.append(sums[dc] + v)
                new_c.append(cnts[dc] + nz)
            return tuple(new_s), tuple(new_c)
        return lax.fori_loop(0, n, body, carry)

    zf = tuple(jnp.zeros((16,), jnp.float32) for _ in range(DC))
    zu = tuple(jnp.zeros((16,), jnp.uint32) for _ in range(DC))

    for r in range(NIB):
        pltpu.async_copy(xa_hbm.at[wid, r], idxa[r], isa[r])
        pltpu.async_copy(xb_hbm.at[wid, r], idxb[r], isb[r])
    for p in range(2):
        wait_idx(xa_hbm, p, idxa, isa)
        fire_gather(p, bufa[p], gsa[p], idxa)
        wait_idx(xb_hbm, p, idxb, isb)
        fire_gather(p, bufb[p], gsb[p], idxb)

    def iter_body(t, _):
        for q in range(4):
            r = 4 * t + q
            p = q % 2
            k2 = (q + 2) % 4

            wait_gather(CHA, bufa[p], gsa[p])
            carry = reduce_chunk(bufa[p], CHA, (zf, zu))

            @pl.when(r + 2 < ROWS_PER_W)
            def _(p=p, k2=k2):
                wait_idx(xa_hbm, k2, idxa, isa)
                fire_gather(k2, bufa[p], gsa[p], idxa)

            @pl.when(r + 4 < ROWS_PER_W)
            def _(q=q, r=r):
                pltpu.async_copy(xa_hbm.at[wid, r + 4], idxa[q], isa[q])

            wait_gather(CHB, bufb[p], gsb[p])
            sums, cnts = reduce_chunk(bufb[p], CHB, carry)

            @pl.when(r + 2 < ROWS_PER_W)
            def _(p=p, k2=k2):
                wait_idx(xb_hbm, k2, idxb, isb)
                fire_gather(k2, bufb[p], gsb[p], idxb)

            @pl.when(r + 4 < ROWS_PER_W)
            def _(q=q, r=r):
                pltpu.async_copy(xb_hbm.at[wid, r + 4], idxb[q], isb[q])

            srow = 4 * lax.rem(t, 2) + q
            for dc in range(DC):
                denom = jnp.maximum(cnts[dc].astype(jnp.float32), 1.0)
                stage[srow, pl.ds(dc * 16, 16)] = sums[dc] / denom

        @pl.when(lax.rem(t, 2) == 1)
        def _():
            base = pl.multiple_of(wid * ROWS_PER_W + 4 * t - 4, 8)
            pltpu.sync_copy(stage, h0_hbm.at[pl.ds(base, 8)])
        return 0

    lax.fori_loop(0, ROWS_PER_W // 4, iter_body, 0)


def _pool(xa, xb, table):
    return pl.kernel(
        _pool_body,
        mesh=plsc.VectorSubcoreMesh(core_axis_name="c", subcore_axis_name="s"),
        out_type=jax.ShapeDtypeStruct((B, D), jnp.float32),
        scratch_types=[
            [pltpu.VMEM((CHA, D), jnp.float32) for _ in range(2)],
            [pltpu.VMEM((CHB, D), jnp.float32) for _ in range(2)],
            [pltpu.VMEM((CHA,), jnp.int32) for _ in range(NIB)],
            [pltpu.VMEM((CHB,), jnp.int32) for _ in range(NIB)],
            pltpu.VMEM((8, D), jnp.float32),
            [pltpu.SemaphoreType.DMA for _ in range(2)],
            [pltpu.SemaphoreType.DMA for _ in range(2)],
            [pltpu.SemaphoreType.DMA for _ in range(NIB)],
            [pltpu.SemaphoreType.DMA for _ in range(NIB)],
        ],
    )(xa, xb, table)


def _mlp_body(h0_ref, W1_ref, b1_ref, W2_ref, b2_ref, out_ref):
    h1 = jnp.dot(h0_ref[...], W1_ref[...], preferred_element_type=jnp.float32)
    h1 = jnp.maximum(h1 + b1_ref[...], 0.0)
    o = jnp.dot(h1, W2_ref[...], preferred_element_type=jnp.float32)
    out_ref[...] = jnp.maximum(o + b2_ref[...], 0.0)


def _mlp(h0, W1, b1, W2, b2):
    bm = 512
    return pl.pallas_call(
        _mlp_body,
        grid=(B // bm,),
        in_specs=[
            pl.BlockSpec((bm, D), lambda i: (i, 0)),
            pl.BlockSpec((D, 128), lambda i: (0, 0)),
            pl.BlockSpec((1, 128), lambda i: (0, 0)),
            pl.BlockSpec((128, NCLASS), lambda i: (0, 0)),
            pl.BlockSpec((1, NCLASS), lambda i: (0, 0)),
        ],
        out_specs=pl.BlockSpec((bm, NCLASS), lambda i: (i, 0)),
        out_shape=jax.ShapeDtypeStruct((B, NCLASS), jnp.float32),
    )(h0, W1, b1.reshape(1, -1), W2, b2.reshape(1, -1))


def kernel(x, table, W1, b1, W2, b2):
    xa = x[:, :CHA].reshape(NW, ROWS_PER_W, CHA)
    xb = x[:, CHA:].reshape(NW, ROWS_PER_W, CHB)
    h0 = _pool(xa, xb, table)
    return _mlp(h0, W1, b1, W2, b2)
